# baseline (speedup 1.0000x reference)
import functools

import jax
import jax.numpy as jnp
from jax import lax
from jax.experimental import pallas as pl
from jax.experimental.pallas import tpu as pltpu
from jax.experimental.pallas import tpu_sc as plsc

_B = 3
_H = 224
_W = 384
_D = 128
_L = 16
_NW = 32
_CPW = 2 * _D // _NW
_HROWS = _H // 2
_HALF = _HROWS * _W


def _pos_body(col_hbm, row_hbm, out_hbm, pat_v, rw_v, buf0_v, buf1_v, sem0, sem1):
    wid = lax.axis_index("s") * 2 + lax.axis_index("c")
    kpr = _W // _L
    bufs = (buf0_v, buf1_v)
    sems = (sem0, sem1)
    inflight = [[], []]

    for slot in range(2 * _CPW):
        t, hh = divmod(slot, 2)
        c = wid * _CPW + t
        nb = slot % 2
        buf, sem = bufs[nb], sems[nb]

        for cp in inflight[nb]:
            cp.wait()
        inflight[nb] = []

        @pl.when(c < _D)
        def _col(c=c, buf=buf):
            if hh == 0:
                pltpu.sync_copy(col_hbm.at[c], pat_v)
            pat = [pat_v[pl.ds(_L * k, _L)] for k in range(kpr)]

            def body(r, carry):
                base = r * _W
                for k in range(kpr):
                    buf[pl.ds(base + _L * k, _L)] = pat[k]
                return carry

            lax.fori_loop(0, _HROWS, body, 0)

        @pl.when(c >= _D)
        def _row(c=c, buf=buf, hh=hh):
            if hh == 0:
                pltpu.sync_copy(row_hbm.at[c - _D], rw_v)

            def body(r, carry):
                v = rw_v[pl.ds((hh * _HROWS + r) * _L, _L)]
                base = r * _W
                for k in range(kpr):
                    buf[pl.ds(base + _L * k, _L)] = v
                return carry

            lax.fori_loop(0, _HROWS, body, 0)

        inflight[nb] = [
            pltpu.async_copy(buf, out_hbm.at[b, c, pl.ds(hh * _HALF, _HALF)], sem)
            for b in range(_B)
        ]

    for pend in inflight:
        for cp in pend:
            cp.wait()


def _tc_kernel(col_ref, row_ref, out_ref):
    d, w = col_ref.shape
    h_blk = row_ref.shape[0]
    col = col_ref[...]
    row = jnp.transpose(row_ref[...], (1, 0))
    out_ref[0, :d] = jnp.broadcast_to(col[:, None, :], (d, h_blk, w))
    out_ref[0, d:] = jnp.broadcast_to(row[:, :, None], (d, h_blk, w))


def kernel(x, row_w, col_w):
    b = x.shape[0]
    h, w = x.shape[-2], x.shape[-1]
    d = row_w.shape[-1]
    col_t = col_w[:w].T
    row_t = jnp.repeat(row_w[:h].T, _L, axis=1)

    mesh = plsc.VectorSubcoreMesh(core_axis_name="c", subcore_axis_name="s")
    run = functools.partial(
        pl.kernel,
        mesh=mesh,
        out_type=jax.ShapeDtypeStruct((_B, 2 * d, h * w), jnp.float32),
        scratch_types=[
            pltpu.VMEM((w,), jnp.float32),
            pltpu.VMEM((h * _L,), jnp.float32),
            pltpu.VMEM((_HALF,), jnp.float32),
            pltpu.VMEM((_HALF,), jnp.float32),
            pltpu.SemaphoreType.DMA,
            pltpu.SemaphoreType.DMA,
        ],
    )(_pos_body)
    sc_out = run(col_t, row_t)

    h_blk = 16
    tc_out = pl.pallas_call(
        _tc_kernel,
        grid=(b - _B, h // h_blk),
        in_specs=[
            pl.BlockSpec((d, w), lambda bi, hi: (0, 0)),
            pl.BlockSpec((h_blk, d), lambda bi, hi: (hi, 0)),
        ],
        out_specs=pl.BlockSpec((1, 2 * d, h_blk, w), lambda bi, hi: (bi, 0, hi, 0)),
        out_shape=jax.ShapeDtypeStruct((b - _B, 2 * d, h, w), jnp.float32),
    )(col_t, row_w[:h])
    return jnp.concatenate([sc_out, tc_out.reshape(b - _B, 2 * d, h * w)], axis=0)


# speedup vs baseline: 2.8468x; 2.8468x over previous
import functools

import jax
import jax.numpy as jnp
from jax import lax
from jax.experimental import pallas as pl
from jax.experimental.pallas import tpu as pltpu
from jax.experimental.pallas import tpu_sc as plsc

_B = 4
_H = 224
_W = 384
_D = 128
_L = 16
_NW = 32
_CPW = 2 * _D // _NW
_NBUF = 4
_HROWS = _H // _NBUF
_HALF = _HROWS * _W


def _pos_body(col_hbm, row_hbm, out_hbm, pat_v, rw_v, b0, b1, b2, b3, s0, s1, s2, s3):
    wid = lax.axis_index("s") * 2 + lax.axis_index("c")
    kpr = _W // _L
    bufs = (b0, b1, b2, b3)
    sems = (s0, s1, s2, s3)
    inflight = [[] for _ in range(_NBUF)]

    for slot in range(_NBUF * _CPW):
        t, hh = divmod(slot, _NBUF)
        c = wid * _CPW + t
        nb = slot % _NBUF
        buf, sem = bufs[nb], sems[nb]

        for cp in inflight[nb]:
            cp.wait()
        inflight[nb] = []

        @pl.when(c < _D)
        def _col(c=c, buf=buf):
            if hh == 0:
                pltpu.sync_copy(col_hbm.at[c], pat_v)
            pat = [pat_v[pl.ds(_L * k, _L)] for k in range(kpr)]

            def body(r, carry):
                base = r * _W
                for k in range(kpr):
                    buf[pl.ds(base + _L * k, _L)] = pat[k]
                return carry

            lax.fori_loop(0, _HROWS, body, 0)

        @pl.when(c >= _D)
        def _row(c=c, buf=buf, hh=hh):
            if hh == 0:
                pltpu.sync_copy(row_hbm.at[c - _D], rw_v)

            def body(r, carry):
                v = rw_v[pl.ds((hh * _HROWS + r) * _L, _L)]
                base = r * _W
                for k in range(kpr):
                    buf[pl.ds(base + _L * k, _L)] = v
                return carry

            lax.fori_loop(0, _HROWS, body, 0)

        inflight[nb] = [
            pltpu.async_copy(buf, out_hbm.at[b, c, pl.ds(hh * _HALF, _HALF)], sem)
            for b in range(_B)
        ]

    for pend in inflight:
        for cp in pend:
            cp.wait()


def kernel(x, row_w, col_w):
    b = x.shape[0]
    h, w = x.shape[-2], x.shape[-1]
    d = row_w.shape[-1]
    col_t = col_w[:w].T
    row_t = jnp.repeat(row_w[:h].T, _L, axis=1)

    mesh = plsc.VectorSubcoreMesh(core_axis_name="c", subcore_axis_name="s")
    run = functools.partial(
        pl.kernel,
        mesh=mesh,
        out_type=jax.ShapeDtypeStruct((b, 2 * d, h * w), jnp.float32),
        scratch_types=[
            pltpu.VMEM((w,), jnp.float32),
            pltpu.VMEM((h * _L,), jnp.float32),
            pltpu.VMEM((_HALF,), jnp.float32),
            pltpu.VMEM((_HALF,), jnp.float32),
            pltpu.VMEM((_HALF,), jnp.float32),
            pltpu.VMEM((_HALF,), jnp.float32),
            pltpu.SemaphoreType.DMA,
            pltpu.SemaphoreType.DMA,
            pltpu.SemaphoreType.DMA,
            pltpu.SemaphoreType.DMA,
        ],
    )(_pos_body)
    return run(col_t, row_t)


# speedup vs baseline: 2.8677x; 1.0074x over previous
import functools

import jax
import jax.numpy as jnp
from jax import lax
from jax.experimental import pallas as pl
from jax.experimental.pallas import tpu as pltpu
from jax.experimental.pallas import tpu_sc as plsc

_B = 4
_H = 224
_W = 384
_D = 128
_L = 16
_NW = 32
_CPW = 2 * _D // _NW
_HROWS = _H // 2
_HALF = _HROWS * _W


def _pos_body(col_hbm, row_hbm, out_hbm, pat_v, rw_v, buf0_v, buf1_v, sem0, sem1):
    wid = lax.axis_index("s") * 2 + lax.axis_index("c")
    kpr = _W // _L
    bufs = (buf0_v, buf1_v)
    sems = (sem0, sem1)
    inflight = [[], []]

    for slot in range(2 * _CPW):
        t, hh = divmod(slot, 2)
        c = wid * _CPW + t
        nb = slot % 2
        buf, sem = bufs[nb], sems[nb]

        for cp in inflight[nb]:
            cp.wait()
        inflight[nb] = []

        @pl.when(c < _D)
        def _col(c=c, buf=buf):
            if hh == 0:
                pltpu.sync_copy(col_hbm.at[c], pat_v)
            pat = [pat_v[pl.ds(_L * k, _L)] for k in range(kpr)]

            def body(r, carry):
                base = r * _W
                for k in range(kpr):
                    buf[pl.ds(base + _L * k, _L)] = pat[k]
                return carry

            lax.fori_loop(0, _HROWS, body, 0)

        @pl.when(c >= _D)
        def _row(c=c, buf=buf, hh=hh):
            if hh == 0:
                pltpu.sync_copy(row_hbm.at[c - _D], rw_v)

            def body(r, carry):
                v = rw_v[pl.ds((hh * _HROWS + r) * _L, _L)]
                base = r * _W
                for k in range(kpr):
                    buf[pl.ds(base + _L * k, _L)] = v
                return carry

            lax.fori_loop(0, _HROWS, body, 0)

        inflight[nb] = [
            pltpu.async_copy(buf, out_hbm.at[b, c, pl.ds(hh * _HALF, _HALF)], sem)
            for b in range(_B)
        ]

    for pend in inflight:
        for cp in pend:
            cp.wait()


def kernel(x, row_w, col_w):
    b = x.shape[0]
    h, w = x.shape[-2], x.shape[-1]
    d = row_w.shape[-1]
    col_t = col_w[:w].T
    row_t = jnp.repeat(row_w[:h].T, _L, axis=1)

    mesh = plsc.VectorSubcoreMesh(core_axis_name="c", subcore_axis_name="s")
    run = functools.partial(
        pl.kernel,
        mesh=mesh,
        out_type=jax.ShapeDtypeStruct((b, 2 * d, h * w), jnp.float32),
        scratch_types=[
            pltpu.VMEM((w,), jnp.float32),
            pltpu.VMEM((h * _L,), jnp.float32),
            pltpu.VMEM((_HALF,), jnp.float32),
            pltpu.VMEM((_HALF,), jnp.float32),
            pltpu.SemaphoreType.DMA,
            pltpu.SemaphoreType.DMA,
        ],
    )(_pos_body)
    return run(col_t, row_t)


# speedup vs baseline: 2.9823x; 1.0400x over previous
import functools

import jax
import jax.numpy as jnp
from jax import lax
from jax.experimental import pallas as pl
from jax.experimental.pallas import tpu as pltpu
from jax.experimental.pallas import tpu_sc as plsc

_B = 4
_H = 224
_W = 384
_D = 128
_L = 16
_NW = 32
_CPW = 2 * _D // _NW
_NBUF = 2
_HROWS = _H // _NBUF
_HALF = _HROWS * _W
_TROW = _W + _H * _L


def _pos_body(tab_hbm, out_hbm, stg_v, buf0_v, buf1_v, ssem, sem0, sem1):
    wid = lax.axis_index("s") * 2 + lax.axis_index("c")
    kpr = _W // _L
    bufs = (buf0_v, buf1_v)
    sems = (sem0, sem1)

    stage = [
        pltpu.async_copy(
            tab_hbm.at[wid * _CPW + t], stg_v.at[pl.ds(t * _TROW, _TROW)], ssem
        )
        for t in range(_CPW)
    ]
    for cp in stage:
        cp.wait()

    inflight = [[] for _ in range(_NBUF)]

    for slot in range(_NBUF * _CPW):
        t, hh = divmod(slot, _NBUF)
        c = wid * _CPW + t
        nb = slot % _NBUF
        buf, sem = bufs[nb], sems[nb]

        for cp in inflight[nb]:
            cp.wait()
        inflight[nb] = []

        pat = [stg_v[pl.ds(t * _TROW + _L * k, _L)] for k in range(kpr)]
        vbase = t * _TROW + _W + hh * _HROWS * _L

        def body(r, carry, pat=pat, buf=buf, vbase=vbase):
            v = stg_v[pl.ds(vbase + r * _L, _L)]
            base = r * _W
            for k in range(kpr):
                buf[pl.ds(base + _L * k, _L)] = pat[k] * v
            return carry

        lax.fori_loop(0, _HROWS, body, 0)

        inflight[nb] = [
            pltpu.async_copy(buf, out_hbm.at[b, c, pl.ds(hh * _HALF, _HALF)], sem)
            for b in range(_B)
        ]

    for pend in inflight:
        for cp in pend:
            cp.wait()


def kernel(x, row_w, col_w):
    b = x.shape[0]
    h, w = x.shape[-2], x.shape[-1]
    d = row_w.shape[-1]
    col_pat = col_w[:w].T
    row_val = jnp.repeat(row_w[:h].T, _L, axis=1)
    ones_pat = jnp.ones((d, w), jnp.float32)
    ones_val = jnp.ones((d, h * _L), jnp.float32)
    tab = jnp.concatenate(
        [
            jnp.concatenate([col_pat, ones_val], axis=1),
            jnp.concatenate([ones_pat, row_val], axis=1),
        ],
        axis=0,
    )

    mesh = plsc.VectorSubcoreMesh(core_axis_name="c", subcore_axis_name="s")
    run = functools.partial(
        pl.kernel,
        mesh=mesh,
        out_type=jax.ShapeDtypeStruct((b, 2 * d, h * w), jnp.float32),
        scratch_types=[
            pltpu.VMEM((_CPW * _TROW,), jnp.float32),
            pltpu.VMEM((_HALF,), jnp.float32),
            pltpu.VMEM((_HALF,), jnp.float32),
            pltpu.SemaphoreType.DMA,
            pltpu.SemaphoreType.DMA,
            pltpu.SemaphoreType.DMA,
        ],
    )(_pos_body)
    return run(tab)
